# Initial kernel scaffold; baseline (speedup 1.0000x reference)
import jax
import jax.numpy as jnp
from jax.experimental import pallas as pl


def kernel(l0_xyz, l0_points, label, params):
    raise NotImplementedError("write your pallas kernel here")


# speedup vs baseline: 1.0119x; 1.0119x over previous
import functools

import jax
import jax.numpy as jnp
import numpy as np
from jax.experimental import pallas as pl
from jax.experimental.pallas import tpu as pltpu

B, N, NCLASSES = 16, 2048, 50
_SEG_CLASSES = [[0,1,2,3],[4,5],[6,7],[8,9,10,11],[12,13,14,15],[16,17,18],[19,20,21],[22,23],[24,25,26,27],[28,29],[30,31,32,33,34,35],[36,37],[38,39,40],[41,42,43],[44,45,46],[47,48,49]]
_m = np.zeros(50, dtype=np.int32)
for _ci, _parts in enumerate(_SEG_CLASSES):
    for _p in _parts:
        _m[_p] = _ci
_SEG_MAP = jnp.asarray(_m)


def _square_distance(a, b):
    return jnp.sum(a*a, -1)[:, :, None] + jnp.sum(b*b, -1)[:, None, :] - 2.0*jnp.einsum('bnc,bmc->bnm', a, b)


def _index_points(points, idx):
    batch = jnp.arange(points.shape[0]).reshape((points.shape[0],) + (1,)*(idx.ndim - 1))
    return points[batch, idx]


def _farthest_point_sample(xyz, M):
    xyz = jax.lax.stop_gradient(xyz)
    b, n, _ = xyz.shape
    def body(i, state):
        dist, idxs, far = state
        c = _index_points(xyz, far[:, None])
        d = jnp.sum((xyz - c)**2, axis=-1)
        dist = jnp.minimum(dist, d)
        idxs = idxs.at[:, i].set(far)
        far = jnp.argmax(dist, axis=-1).astype(jnp.int32)
        return (dist, idxs, far)
    state = (jnp.full((b, n), 1e10, jnp.float32), jnp.zeros((b, M), jnp.int32), jnp.zeros((b,), jnp.int32))
    state = jax.lax.fori_loop(0, M, body, state)
    return state[1]


def _ball_query(radius, K, xyz, new_xyz):
    b, n = xyz.shape[0], xyz.shape[1]
    m = new_xyz.shape[1]
    sq = jax.lax.stop_gradient(_square_distance(new_xyz, xyz))
    gidx = jnp.broadcast_to(jnp.arange(n, dtype=jnp.int32), (b, m, n))
    gidx = jnp.where(sq > radius*radius, n, gidx)
    gidx = jnp.sort(gidx, axis=-1)[:, :, :K]
    first = gidx[:, :, :1]
    return jnp.where(gidx == n, jnp.broadcast_to(first, gidx.shape), gidx)


def _bn_relu(x, layer):
    return jax.nn.relu(x @ layer['W'] * layer['g'] + layer['b'])


def _sa_module(xyz, points, M, radius, K, layers, group_all):
    if group_all:
        new_xyz = jnp.zeros((xyz.shape[0], 1, 3), jnp.float32)
        new_points = jnp.concatenate([xyz[:, None], points[:, None]], axis=-1)
    else:
        fps_idx = _farthest_point_sample(xyz, M)
        new_xyz = _index_points(xyz, fps_idx)
        gidx = _ball_query(radius, K, xyz, new_xyz)
        grouped_xyz = _index_points(xyz, gidx) - new_xyz[:, :, None, :]
        grouped_pts = _index_points(points, gidx)
        new_points = jnp.concatenate([grouped_xyz, grouped_pts], axis=-1)
    for layer in layers:
        new_points = _bn_relu(new_points, layer)
    return new_xyz, jnp.max(new_points, axis=2)


def _fp_module(xyz1, xyz2, points1, points2, layers):
    b, n = xyz1.shape[0], xyz1.shape[1]
    m = xyz2.shape[1]
    if m == 1:
        interp = jnp.broadcast_to(points2, (b, n, points2.shape[-1]))
    else:
        d = _square_distance(xyz1, xyz2)
        negd, idx = jax.lax.top_k(-d, 3)
        d3 = jnp.maximum(-negd, 0.0)
        w = 1.0 / (d3 + 1e-8)
        w = w / jnp.sum(w, axis=-1, keepdims=True)
        interp = jnp.sum(_index_points(points2, idx) * w[..., None], axis=2)
    new = jnp.concatenate([interp, points1], axis=-1)
    for layer in layers:
        new = _bn_relu(new, layer)
    return new


def _clip_model(xyz, p):
    h = jax.nn.relu(jnp.einsum('bnc,cd->bnd', xyz, p['clip_W1']))
    feat = jnp.mean(h, axis=1) @ p['clip_W2']
    feat = feat / (jnp.linalg.norm(feat, axis=-1, keepdims=True) + 1e-8)
    return feat, 100.0 * feat @ p['clip_Wtext'].T


def _adapter_fn(x, p, alpha=0.5):
    h = jax.nn.relu(jax.nn.relu(x @ p['ada_W1']) @ p['ada_W2'])
    return alpha * h + (1.0 - alpha) * x


def _cosine_loss(A, Bv, t=1.0):
    num = jnp.sum(A * Bv, axis=-1)
    den = jnp.linalg.norm(A, axis=-1) * jnp.linalg.norm(Bv, axis=-1) + 1e-8
    return jnp.mean(jnp.maximum(t - num / den, 0.0))


def _ce_loss(logits, labels):
    lp = jax.nn.log_softmax(logits, axis=-1)
    return -jnp.mean(jnp.take_along_axis(lp, labels[:, None], axis=1))


_TR = 2048


def _head_body(x_ref, *refs):
    (w1, g1, b1, w2, g2, b2, w3, g3, b3,
     wc, gc, bc, wcls, bcls, o_ref, f_ref) = refs
    x = x_ref[...]
    h = jnp.maximum(jnp.dot(x, w1[...]) * g1[...] + b1[...], 0.0)
    h = jnp.maximum(jnp.dot(h, w2[...]) * g2[...] + b2[...], 0.0)
    h = jnp.maximum(jnp.dot(h, w3[...]) * g3[...] + b3[...], 0.0)
    f_ref[...] = h
    h = jnp.maximum(jnp.dot(h, wc[...]) * gc[...] + bc[...], 0.0)
    o_ref[...] = jnp.dot(h, wcls[...]) + bcls[...]


@functools.partial(jax.jit, static_argnames=())
def _head_kernel(x, p):
    rows = x.shape[0]
    cin = x.shape[1]
    w1 = p['fp3'][0]['W']
    full = lambda s: pl.BlockSpec(s, lambda i: (0, 0))
    args = []
    specs = [pl.BlockSpec((_TR, cin), lambda i: (i, 0))]
    for layer in (p['fp3'][0], p['fp3'][1], p['fp3'][2], p['conv1']):
        args += [layer['W'], layer['g'].reshape(1, -1), layer['b'].reshape(1, -1)]
        specs += [full(layer['W'].shape), full((1, layer['W'].shape[1])),
                  full((1, layer['W'].shape[1]))]
    args += [p['cls_W'], p['cls_b'].reshape(1, -1)]
    specs += [full(p['cls_W'].shape), full((1, NCLASSES))]
    out, feat = pl.pallas_call(
        _head_body,
        grid=(rows // _TR,),
        in_specs=specs,
        out_specs=[pl.BlockSpec((_TR, NCLASSES), lambda i: (i, 0)),
                   pl.BlockSpec((_TR, 128), lambda i: (i, 0))],
        out_shape=[jax.ShapeDtypeStruct((rows, NCLASSES), jnp.float32),
                   jax.ShapeDtypeStruct((rows, 128), jnp.float32)],
    )(x, *args)
    return out, feat


def kernel(l0_xyz, l0_points, label, params):
    p = params
    new_label = _SEG_MAP[label[:, 0]]
    image_feat_1, logits_1 = _clip_model(l0_xyz, p)
    l1_xyz, l1_points = _sa_module(l0_xyz, l0_points, 512, 0.2, 32, p['sa1'], False)
    pf1 = jnp.mean(l1_points, axis=1)
    pf1 = _bn_relu(pf1, p['fc1a'])
    pf1 = _bn_relu(pf1, p['fc1b'])
    pf1 = _adapter_fn(pf1, p)
    ls1 = _cosine_loss(image_feat_1, pf1, 0.8)
    image_feat_2, logits_2 = _clip_model(l1_xyz, p)
    l2_xyz, l2_points = _sa_module(l1_xyz, l1_points, 128, 0.4, 64, p['sa2'], False)
    pf2 = jnp.mean(l2_points, axis=1)
    pf2 = _bn_relu(pf2, p['fc2'])
    pf2 = _adapter_fn(pf2, p)
    ls2 = _cosine_loss(image_feat_2, pf2, 0.8)
    image_feat_3, logits_3 = _clip_model(l2_xyz, p)
    l3_xyz, l3_points = _sa_module(l2_xyz, l2_points, None, None, None, p['sa3'], True)
    pf3 = l3_points.reshape(l3_points.shape[0], -1)
    pf3 = _bn_relu(pf3, p['cls_fc1'])
    pf3 = _adapter_fn(pf3, p)
    ls3 = _cosine_loss(image_feat_3, pf3, 0.8)
    l2p = _fp_module(l2_xyz, l3_xyz, l2_points, l3_points, p['fp1'])
    l1p = _fp_module(l1_xyz, l2_xyz, l1_points, l2p, p['fp2'])

    d = _square_distance(l0_xyz, l1_xyz)
    negd, idx = jax.lax.top_k(-d, 3)
    d3 = jnp.maximum(-negd, 0.0)
    w = 1.0 / (d3 + 1e-8)
    w = w / jnp.sum(w, axis=-1, keepdims=True)
    interp = jnp.sum(_index_points(l1p, idx) * w[..., None], axis=2)
    points1 = jnp.concatenate([l0_points, l0_xyz], axis=-1)
    x = jnp.concatenate([interp, points1], axis=-1).reshape(B * N, 134)

    net_flat, l0p_flat = _head_kernel(x, p)
    net = net_flat.reshape(B, N, NCLASSES)
    l0p = l0p_flat.reshape(B, N, 128)
    pf4 = jnp.mean(l0p, axis=1)
    pf4 = _bn_relu(pf4, p['fc1a'])
    pf4 = _bn_relu(pf4, p['fc1b'])
    pf4 = _adapter_fn(pf4, p)
    ls4 = _cosine_loss(image_feat_1, pf4, 0.8)

    net = jnp.transpose(net, (0, 2, 1))
    cl1 = _ce_loss(logits_1, new_label)
    cl2 = _ce_loss(logits_2, new_label)
    cl3 = _ce_loss(logits_3, new_label)
    loss_cur = ls1 + ls2 + ls3 + ls4 + cl1 + cl2 + cl3
    return loss_cur, net
